# baseline (speedup 1.0000x reference)
import functools

import jax
import jax.numpy as jnp
from jax import lax
from jax.experimental import pallas as pl
from jax.experimental.pallas import tpu as pltpu
from jax.experimental.pallas import tpu_sc as plsc

N_USERS = 25000
N_ITEMS = 25000
N_NODES = N_USERS + N_ITEMS
D = 64
H = 32
E = 800000
NL = 3

NT = 16
SUB = 128
NSUB = 6
BLK = NSUB * SUB
GPS = SUB // 16
G = 2
NSUP = 33
BPT = G * NSUP
EP = NT * BPT * BLK
NBLK = NT * BPT
RPT = N_NODES // NT


def _prop_body(emb0, colb, rowb, wb, zer, e1, e2, e3,
               colv, rowv, wv, rowsbuf, acc, gsem, ssem, isem):
  c = lax.axis_index("c")
  s = lax.axis_index("s")
  coff = c * N_NODES

  def zero_acc():
    pltpu.sync_copy(zer.at[pl.ds(s * RPT, RPT)], acc.at[pl.ds(s * RPT, RPT)])

  zero_acc()
  plsc.subcore_barrier()

  outs = (e1, e2, e3)
  for layer in range(NL):
    src = emb0 if layer == 0 else outs[layer - 1]
    dst = outs[layer]

    @pl.loop(0, NSUP)
    def _(u):
      base = s * BPT + u * G
      i1 = pltpu.async_copy(colb.at[pl.ds(base, G)], colv, isem)
      i2 = pltpu.async_copy(rowb.at[pl.ds(base, G)], rowv, isem)
      i3 = pltpu.async_copy(wb.at[pl.ds(base, G)], wv, isem)
      i1.wait()
      i2.wait()
      i3.wait()

      @pl.loop(0, G)
      def _(b):
        for j in range(NSUB):
          for q in range(SUB // 16):
            sl = (b, j, pl.ds(q * 16, 16))
            colv[sl] = colv[sl] + coff
        gd = [pltpu.async_copy(src.at[colv.at[b].at[j]], rowsbuf.at[j], gsem)
              for j in range(NSUB)]
        sd = []
        for j in range(NSUB):
          gd[j].wait()

          @pl.loop(0, GPS)
          def _(g, j=j):
            w16 = wv[b, pl.ds(j * SUB + g * 16, 16)]
            for k in range(16):
              wk = lax.broadcast_in_dim(w16[k], (16,), ())
              i = g * 16 + k
              lo = (j, i, pl.ds(0, 16))
              hi = (j, i, pl.ds(16, 16))
              rowsbuf[lo] = rowsbuf[lo] * wk
              rowsbuf[hi] = rowsbuf[hi] * wk

          sd.append(pltpu.async_copy(rowsbuf.at[j], acc.at[rowv.at[b].at[j]],
                                     ssem, add=True))
        for d_ in sd:
          d_.wait()

    plsc.subcore_barrier()
    pltpu.sync_copy(acc.at[pl.ds(s * RPT, RPT)],
                    dst.at[pl.ds(coff + s * RPT, RPT)])
    if layer < NL - 1:
      zero_acc()
    plsc.subcore_barrier()


_prop = pl.kernel(
    _prop_body,
    out_type=(jax.ShapeDtypeStruct((2 * N_NODES, H), jnp.float32),) * 3,
    mesh=plsc.VectorSubcoreMesh(core_axis_name="c", subcore_axis_name="s"),
    scratch_types=[
        pltpu.VMEM((G, NSUB, SUB), jnp.int32),
        pltpu.VMEM((G, NSUB, SUB), jnp.int32),
        pltpu.VMEM((G, BLK), jnp.float32),
        pltpu.VMEM((NSUB, SUB, H), jnp.float32),
        pltpu.VMEM_SHARED((N_NODES, H), jnp.float32),
        pltpu.SemaphoreType.DMA,
        pltpu.SemaphoreType.DMA,
        pltpu.SemaphoreType.DMA,
    ],
    compiler_params=pltpu.CompilerParams(
        use_tc_tiling_on_sc=False, needs_layout_passes=False),
)


def _mean_body(a, b, c, d, o):
  o[...] = 0.25 * (a[...] + b[...] + c[...] + d[...])


_MROWS = 2 * N_NODES * H // 128
_MB = 1000

_mean = pl.pallas_call(
    _mean_body,
    grid=(_MROWS // _MB,),
    in_specs=[pl.BlockSpec((_MB, 128), lambda i: (i, 0))] * 4,
    out_specs=pl.BlockSpec((_MB, 128), lambda i: (i, 0)),
    out_shape=jax.ShapeDtypeStruct((_MROWS, 128), jnp.float32),
)


@jax.jit
def kernel(edge_index, edge_weight, user_emb, item_emb):
  all_emb = jnp.concatenate([user_emb, item_emb], axis=0)
  emb0 = jnp.concatenate([all_emb[:, :H], all_emb[:, H:]], axis=0)
  pad = EP - E
  col = jnp.concatenate([edge_index[1], jnp.zeros((pad,), jnp.int32)])
  row = jnp.concatenate([edge_index[0], jnp.zeros((pad,), jnp.int32)])
  w = jnp.concatenate([edge_weight, jnp.zeros((pad,), jnp.float32)])
  colb = col.reshape(NBLK, NSUB, SUB)
  rowb = row.reshape(NBLK, NSUB, SUB)
  wb = w.reshape(NBLK, BLK)
  zer = jnp.zeros((N_NODES, H), jnp.float32)
  e1, e2, e3 = _prop(emb0, colb, rowb, wb, zer)
  m = _mean(emb0.reshape(_MROWS, 128), e1.reshape(_MROWS, 128),
            e2.reshape(_MROWS, 128), e3.reshape(_MROWS, 128))
  m = m.reshape(2 * N_NODES, H)
  final = jnp.concatenate([m[:N_NODES], m[N_NODES:]], axis=1)
  return final[:N_USERS], final[N_USERS:]


# speedup vs baseline: 1.3606x; 1.3606x over previous
import functools

import jax
import jax.numpy as jnp
from jax import lax
from jax.experimental import pallas as pl
from jax.experimental.pallas import tpu as pltpu
from jax.experimental.pallas import tpu_sc as plsc

N_USERS = 25000
N_ITEMS = 25000
N_NODES = N_USERS + N_ITEMS
D = 64
H = 32
E = 800000
NL = 3

NT = 16
SUB = 128
NSUB = 4
BLK = NSUB * SUB
GPS = SUB // 16
G = 7
NSUP = 14
BPT = G * NSUP
EP = NT * BPT * BLK
NBLK = NT * BPT
RPT = N_NODES // NT


def _prop_body(emb0, colb, rowb, wb, zer, e1, e2, e3,
               colv, rowv, wv, rowsbuf, acc, gsem, isem, *ssems):
  c = lax.axis_index("c")
  s = lax.axis_index("s")
  coff = c * N_NODES

  def zero_acc():
    pltpu.sync_copy(zer.at[pl.ds(s * RPT, RPT)], acc.at[pl.ds(s * RPT, RPT)])

  zero_acc()
  plsc.subcore_barrier()

  outs = (e1, e2, e3)
  for layer in range(NL):
    src = emb0 if layer == 0 else outs[layer - 1]
    dst = outs[layer]

    def drain_scatter(j):
      pltpu.make_async_copy(emb0.at[pl.ds(0, SUB)], rowsbuf.at[j],
                            ssems[j]).wait()

    @pl.loop(0, NSUP)
    def _(u):
      base = s * BPT + u * G
      @pl.when(u > 0)
      def _():
        for j in range(NSUB):
          drain_scatter(j)
      i1 = pltpu.async_copy(colb.at[pl.ds(base, G)], colv, isem)
      i2 = pltpu.async_copy(rowb.at[pl.ds(base, G)], rowv, isem)
      i3 = pltpu.async_copy(wb.at[pl.ds(base, G)], wv, isem)
      i1.wait()
      i2.wait()
      i3.wait()

      @pl.loop(0, G)
      def _(b):
        for j in range(NSUB):
          for q in range(SUB // 16):
            sl = (b, j, pl.ds(q * 16, 16))
            colv[sl] = colv[sl] + coff
        gd = []
        for j in range(NSUB):
          @pl.when(b > 0)
          def _(j=j):
            drain_scatter(j)
          gd.append(pltpu.async_copy(src.at[colv.at[b].at[j]], rowsbuf.at[j],
                                     gsem))
        for j in range(NSUB):
          gd[j].wait()

          @pl.loop(0, GPS)
          def _(g, j=j):
            w16 = wv[b, pl.ds(j * SUB + g * 16, 16)]
            for k in range(16):
              wk = lax.broadcast_in_dim(w16[k], (16,), ())
              i = g * 16 + k
              lo = (j, i, pl.ds(0, 16))
              hi = (j, i, pl.ds(16, 16))
              rowsbuf[lo] = rowsbuf[lo] * wk
              rowsbuf[hi] = rowsbuf[hi] * wk

          pltpu.async_copy(rowsbuf.at[j], acc.at[rowv.at[b].at[j]],
                           ssems[j], add=True)

    for j in range(NSUB):
      drain_scatter(j)
    plsc.subcore_barrier()
    pltpu.sync_copy(acc.at[pl.ds(s * RPT, RPT)],
                    dst.at[pl.ds(coff + s * RPT, RPT)])
    if layer < NL - 1:
      zero_acc()
    plsc.subcore_barrier()


_prop = pl.kernel(
    _prop_body,
    out_type=(jax.ShapeDtypeStruct((2 * N_NODES, H), jnp.float32),) * 3,
    mesh=plsc.VectorSubcoreMesh(core_axis_name="c", subcore_axis_name="s"),
    scratch_types=[
        pltpu.VMEM((G, NSUB, SUB), jnp.int32),
        pltpu.VMEM((G, NSUB, SUB), jnp.int32),
        pltpu.VMEM((G, BLK), jnp.float32),
        pltpu.VMEM((NSUB, SUB, H), jnp.float32),
        pltpu.VMEM_SHARED((N_NODES, H), jnp.float32),
        pltpu.SemaphoreType.DMA,
        pltpu.SemaphoreType.DMA,
        pltpu.SemaphoreType.DMA,
        pltpu.SemaphoreType.DMA,
        pltpu.SemaphoreType.DMA,
        pltpu.SemaphoreType.DMA,
    ],
    compiler_params=pltpu.CompilerParams(
        use_tc_tiling_on_sc=False, needs_layout_passes=False),
)


def _mean_body(a, b, c, d, o):
  o[...] = 0.25 * (a[...] + b[...] + c[...] + d[...])


_MROWS = 2 * N_NODES * H // 128
_MB = 1000

_mean = pl.pallas_call(
    _mean_body,
    grid=(_MROWS // _MB,),
    in_specs=[pl.BlockSpec((_MB, 128), lambda i: (i, 0))] * 4,
    out_specs=pl.BlockSpec((_MB, 128), lambda i: (i, 0)),
    out_shape=jax.ShapeDtypeStruct((_MROWS, 128), jnp.float32),
)


@jax.jit
def kernel(edge_index, edge_weight, user_emb, item_emb):
  all_emb = jnp.concatenate([user_emb, item_emb], axis=0)
  emb0 = jnp.concatenate([all_emb[:, :H], all_emb[:, H:]], axis=0)
  pad = EP - E
  col = jnp.concatenate([edge_index[1], jnp.zeros((pad,), jnp.int32)])
  row = jnp.concatenate([edge_index[0], jnp.zeros((pad,), jnp.int32)])
  w = jnp.concatenate([edge_weight, jnp.zeros((pad,), jnp.float32)])
  colb = col.reshape(NBLK, NSUB, SUB)
  rowb = row.reshape(NBLK, NSUB, SUB)
  wb = w.reshape(NBLK, BLK)
  zer = jnp.zeros((N_NODES, H), jnp.float32)
  e1, e2, e3 = _prop(emb0, colb, rowb, wb, zer)
  m = _mean(emb0.reshape(_MROWS, 128), e1.reshape(_MROWS, 128),
            e2.reshape(_MROWS, 128), e3.reshape(_MROWS, 128))
  m = m.reshape(2 * N_NODES, H)
  final = jnp.concatenate([m[:N_NODES], m[N_NODES:]], axis=1)
  return final[:N_USERS], final[N_USERS:]
